# Initial kernel scaffold; baseline (speedup 1.0000x reference)
import jax
import jax.numpy as jnp
from jax.experimental import pallas as pl


def kernel(user_ids, user_tags_idx, item_ids, content_features, user_table, tag_table, item_table, tp_W, tp_b, tp_g, tp_beta, uf_W, uf_b, uf_g, uf_beta, cf_W, cf_b, cf_g, cf_beta, if_W, if_b, if_g, if_beta, m1_W, m1_b, m2_W, m2_b, m3_W, m3_b, p_W, p_b):
    raise NotImplementedError("write your pallas kernel here")


# speedup vs baseline: 4.9746x; 4.9746x over previous
import functools

import jax
import jax.numpy as jnp
from jax import lax
from jax.experimental import pallas as pl
from jax.experimental.pallas import tpu as pltpu
from jax.experimental.pallas import tpu_sc as plsc

B = 16384
D = 128
UF = 64
CD = 128
H = 20
NC = 2
NS = 16
NW = NC * NS
S = B // NW
CHUNK = 128
N_IDC = S // CHUNK
N_TAG = S * H // CHUNK


def _sc_gather_body(tags3, uids3, iids3, pos3, user_table, tag_table,
                    item_table, uid_out, tsum_out, iid_out,
                    tidx_v, pos_v, idx_v, rows_v, acc_sh, sem):
    cid = lax.axis_index("c")
    sid = lax.axis_index("s")
    wid = sid * NC + cid
    base = wid * S
    slab = sid * S

    def _zero_row(i, carry):
        for cc in range(D // 16):
            rows_v[i, pl.ds(cc * 16, 16)] = jnp.zeros((16,), jnp.float32)
        return carry
    lax.fori_loop(0, CHUNK, _zero_row, 0)
    for c in range(S // CHUNK):
        pltpu.sync_copy(rows_v, acc_sh.at[pl.ds(slab + c * CHUNK, CHUNK)])

    for ids3, table, out in ((uids3, user_table, uid_out),
                             (iids3, item_table, iid_out)):
        pltpu.sync_copy(ids3.at[wid], idx_v)
        for c in range(N_IDC):
            pltpu.async_copy(table.at[idx_v.at[c]], rows_v, sem).wait()
            pltpu.sync_copy(rows_v, out.at[pl.ds(base + c * CHUNK, CHUNK)])

    pltpu.sync_copy(tags3.at[wid], tidx_v)
    pltpu.sync_copy(pos3.at[sid], pos_v)

    def _tag_chunk(c, carry):
        pltpu.async_copy(tag_table.at[tidx_v.at[c]], rows_v, sem).wait()
        pltpu.sync_copy(rows_v, acc_sh.at[pos_v.at[c]], add=True)
        return carry
    lax.fori_loop(0, N_TAG, _tag_chunk, 0)

    pltpu.sync_copy(acc_sh.at[pl.ds(slab, S)], tsum_out.at[pl.ds(base, S)])


def _sc_gather(user_ids, user_tags_idx, item_ids, user_table, tag_table,
               item_table):
    tags3 = user_tags_idx.reshape(NW, N_TAG, CHUNK)
    uids3 = user_ids.reshape(NW, N_IDC, CHUNK)
    iids3 = item_ids.reshape(NW, N_IDC, CHUNK)
    pos3 = (jnp.arange(NS, dtype=jnp.int32)[:, None] * S
            + jnp.repeat(jnp.arange(S, dtype=jnp.int32), H)[None, :]
            ).reshape(NS, N_TAG, CHUNK)
    mesh = plsc.VectorSubcoreMesh(core_axis_name="c", subcore_axis_name="s")
    fn = pl.kernel(
        _sc_gather_body,
        out_type=[jax.ShapeDtypeStruct((B, D), jnp.float32) for _ in range(3)],
        mesh=mesh,
        scratch_types=[
            pltpu.VMEM((N_TAG, CHUNK), jnp.int32),
            pltpu.VMEM((N_TAG, CHUNK), jnp.int32),
            pltpu.VMEM((N_IDC, CHUNK), jnp.int32),
            pltpu.VMEM((CHUNK, D), jnp.float32),
            pltpu.VMEM_SHARED((NS * S, D), jnp.float32),
            pltpu.SemaphoreType.DMA,
        ],
    )
    return fn(tags3, uids3, iids3, pos3, user_table, tag_table, item_table)


def _dot(a, w):
    return lax.dot_general(a, w, (((1,), (1,)), ((), ())),
                           preferred_element_type=jnp.float32,
                           precision=lax.Precision.HIGHEST)


def _ln_aff(x, g, b, eps=1e-5):
    m = jnp.mean(x, axis=-1, keepdims=True)
    v = jnp.mean((x - m) ** 2, axis=-1, keepdims=True)
    return (x - m) * lax.rsqrt(v + eps) * g + b


def _tc_body(uid, tsum, iid, cf, tp_W, tp_b, tp_g, tp_beta, uf_W, uf_b, uf_g,
             uf_beta, cf_W, cf_b, cf_g, cf_beta, if_W, if_b, if_g, if_beta,
             m1_W, m1_b, m2_W, m2_b, m3_W, m3_b, p_W, p_b, out_ref):
    tag = tsum[...] * (1.0 / H)
    t = _ln_aff(jax.nn.relu(_dot(tag, tp_W[...]) + tp_b[...]),
                tp_g[...], tp_beta[...])
    ufW = uf_W[...]
    ue = _ln_aff(jax.nn.relu(_dot(uid[...], ufW[:, :D]) + _dot(t, ufW[:, D:])
                             + uf_b[...]), uf_g[...], uf_beta[...])
    ce = _ln_aff(jax.nn.relu(_dot(cf[...], cf_W[...]) + cf_b[...]),
                 cf_g[...], cf_beta[...])
    ifW = if_W[...]
    ie = _ln_aff(jax.nn.relu(_dot(iid[...], ifW[:, :D]) + _dot(ce, ifW[:, D:])
                             + if_b[...]), if_g[...], if_beta[...])
    m1W = m1_W[...]
    h = jax.nn.relu(_dot(ue, m1W[:, :UF]) + _dot(ie, m1W[:, UF:]) + m1_b[...])
    h = jax.nn.relu(_dot(h, m2_W[...]) + m2_b[...])
    h = jax.nn.relu(_dot(h, m3_W[...]) + m3_b[...])
    logit = jnp.sum(h * p_W[...], axis=1, keepdims=True) + p_b[0, 0]
    out_ref[...] = jax.nn.sigmoid(logit)


def kernel(user_ids, user_tags_idx, item_ids, content_features, user_table,
           tag_table, item_table, tp_W, tp_b, tp_g, tp_beta, uf_W, uf_b, uf_g,
           uf_beta, cf_W, cf_b, cf_g, cf_beta, if_W, if_b, if_g, if_beta,
           m1_W, m1_b, m2_W, m2_b, m3_W, m3_b, p_W, p_b):
    uid, tsum, iid = _sc_gather(user_ids, user_tags_idx, item_ids,
                                user_table, tag_table, item_table)

    BM = 1024
    grid = (B // BM,)
    row = pl.BlockSpec((BM, D), lambda i: (i, 0))
    full = lambda a: pl.BlockSpec(a.shape, lambda i: tuple(0 for _ in a.shape))
    weights = [tp_W, tp_b.reshape(1, -1), tp_g.reshape(1, -1),
               tp_beta.reshape(1, -1), uf_W, uf_b.reshape(1, -1),
               uf_g.reshape(1, -1), uf_beta.reshape(1, -1), cf_W,
               cf_b.reshape(1, -1), cf_g.reshape(1, -1),
               cf_beta.reshape(1, -1), if_W, if_b.reshape(1, -1),
               if_g.reshape(1, -1), if_beta.reshape(1, -1), m1_W,
               m1_b.reshape(1, -1), m2_W, m2_b.reshape(1, -1), m3_W,
               m3_b.reshape(1, -1), p_W, p_b.reshape(1, -1)]
    out = pl.pallas_call(
        _tc_body,
        grid=grid,
        in_specs=[row, row, row, row] + [full(w) for w in weights],
        out_specs=pl.BlockSpec((BM, 1), lambda i: (i, 0)),
        out_shape=jax.ShapeDtypeStruct((B, 1), jnp.float32),
    )(uid, tsum, iid, content_features, *weights)
    return out.reshape(B)


# speedup vs baseline: 9.8708x; 1.9842x over previous
import functools

import jax
import jax.numpy as jnp
from jax import lax
from jax.experimental import pallas as pl
from jax.experimental.pallas import tpu as pltpu
from jax.experimental.pallas import tpu_sc as plsc

B = 16384
D = 128
UF = 64
CD = 128
H = 20
NC = 2
NS = 16
NW = NC * NS
S = B // NW
CHUNK = 128
N_IDC = S // CHUNK
N_TAG = S * H // CHUNK


def _sc_gather_body(tags3, uids3, iids3, pos3, user_table, tag_table,
                    item_table, uid_out, tsum_out, iid_out,
                    tidx_v, pos_v, idx_v, rows_a, rows_b, acc_sh,
                    sem_a, sem_b):
    cid = lax.axis_index("c")
    sid = lax.axis_index("s")
    wid = sid * NC + cid
    base = wid * S
    slab = sid * S
    bufs = (rows_a, rows_b)
    sems = (sem_a, sem_b)

    def _zero_row(i, carry):
        for cc in range(D // 16):
            rows_a[i, pl.ds(cc * 16, 16)] = jnp.zeros((16,), jnp.float32)
        return carry
    lax.fori_loop(0, CHUNK, _zero_row, 0)
    for c in range(S // CHUNK):
        pltpu.sync_copy(rows_a, acc_sh.at[pl.ds(slab + c * CHUNK, CHUNK)])

    for ids3, table, out in ((uids3, user_table, uid_out),
                             (iids3, item_table, iid_out)):
        pltpu.sync_copy(ids3.at[wid], idx_v)
        cps = [pltpu.async_copy(table.at[idx_v.at[0]], bufs[0], sems[0])]
        for c in range(N_IDC):
            if c + 1 < N_IDC:
                cps.append(pltpu.async_copy(table.at[idx_v.at[c + 1]],
                                            bufs[(c + 1) % 2],
                                            sems[(c + 1) % 2]))
            cps[c].wait()
            pltpu.sync_copy(bufs[c % 2], out.at[pl.ds(base + c * CHUNK, CHUNK)])

    pltpu.sync_copy(tags3.at[wid], tidx_v)
    pltpu.sync_copy(pos3.at[sid], pos_v)

    pltpu.async_copy(tag_table.at[tidx_v.at[0]], rows_a, sem_a)

    def _tag_pair(i, carry):
        c = 2 * i
        pltpu.async_copy(tag_table.at[tidx_v.at[c + 1]], rows_b, sem_b)
        pltpu.make_async_copy(tag_table.at[tidx_v.at[c]], rows_a, sem_a).wait()
        pltpu.sync_copy(rows_a, acc_sh.at[pos_v.at[c]], add=True)

        @pl.when(i < N_TAG // 2 - 1)
        def _():
            pltpu.async_copy(tag_table.at[tidx_v.at[c + 2]], rows_a, sem_a)

        pltpu.make_async_copy(tag_table.at[tidx_v.at[c + 1]], rows_b,
                              sem_b).wait()
        pltpu.sync_copy(rows_b, acc_sh.at[pos_v.at[c + 1]], add=True)
        return carry
    lax.fori_loop(0, N_TAG // 2, _tag_pair, 0)

    pltpu.sync_copy(acc_sh.at[pl.ds(slab, S)], tsum_out.at[pl.ds(base, S)])


def _sc_gather(user_ids, user_tags_idx, item_ids, user_table, tag_table,
               item_table):
    tags3 = user_tags_idx.reshape(NW, N_TAG, CHUNK)
    uids3 = user_ids.reshape(NW, N_IDC, CHUNK)
    iids3 = item_ids.reshape(NW, N_IDC, CHUNK)
    pos3 = (jnp.arange(NS, dtype=jnp.int32)[:, None] * S
            + jnp.repeat(jnp.arange(S, dtype=jnp.int32), H)[None, :]
            ).reshape(NS, N_TAG, CHUNK)
    mesh = plsc.VectorSubcoreMesh(core_axis_name="c", subcore_axis_name="s")
    fn = pl.kernel(
        _sc_gather_body,
        out_type=[jax.ShapeDtypeStruct((B, D), jnp.float32) for _ in range(3)],
        mesh=mesh,
        scratch_types=[
            pltpu.VMEM((N_TAG, CHUNK), jnp.int32),
            pltpu.VMEM((N_TAG, CHUNK), jnp.int32),
            pltpu.VMEM((N_IDC, CHUNK), jnp.int32),
            pltpu.VMEM((CHUNK, D), jnp.float32),
            pltpu.VMEM((CHUNK, D), jnp.float32),
            pltpu.VMEM_SHARED((NS * S, D), jnp.float32),
            pltpu.SemaphoreType.DMA,
            pltpu.SemaphoreType.DMA,
        ],
    )
    return fn(tags3, uids3, iids3, pos3, user_table, tag_table, item_table)


def _dot(a, w):
    return lax.dot_general(a, w, (((1,), (0,)), ((), ())),
                           preferred_element_type=jnp.float32)


def _ln_aff(x, g, b, eps=1e-5):
    m = jnp.mean(x, axis=-1, keepdims=True)
    v = jnp.mean((x - m) ** 2, axis=-1, keepdims=True)
    return (x - m) * lax.rsqrt(v + eps) * g + b


def _tc_body(uid, tsum, iid, cf, tp_W, tp_b, tp_g, tp_beta, uf_W, uf_b, uf_g,
             uf_beta, cf_W, cf_b, cf_g, cf_beta, if_W, if_b, if_g, if_beta,
             m1_W, m1_b, m2_W, m2_b, m3_W, m3_b, p_W, p_b, out_ref):
    tag = tsum[...] * (1.0 / H)
    t = _ln_aff(jax.nn.relu(_dot(tag, tp_W[...]) + tp_b[...]),
                tp_g[...], tp_beta[...])
    ufW = uf_W[...]
    ue = _ln_aff(jax.nn.relu(_dot(uid[...], ufW[:D]) + _dot(t, ufW[D:])
                             + uf_b[...]), uf_g[...], uf_beta[...])
    ce = _ln_aff(jax.nn.relu(_dot(cf[...], cf_W[...]) + cf_b[...]),
                 cf_g[...], cf_beta[...])
    ifW = if_W[...]
    ie = _ln_aff(jax.nn.relu(_dot(iid[...], ifW[:D]) + _dot(ce, ifW[D:])
                             + if_b[...]), if_g[...], if_beta[...])
    m1W = m1_W[...]
    h = jax.nn.relu(_dot(ue, m1W[:UF]) + _dot(ie, m1W[UF:]) + m1_b[...])
    h = jax.nn.relu(_dot(h, m2_W[...]) + m2_b[...])
    h = jax.nn.relu(_dot(h, m3_W[...]) + m3_b[...])
    logit = jnp.sum(h * p_W[...], axis=1, keepdims=True) + p_b[0, 0]
    out_ref[...] = jax.nn.sigmoid(logit)


def kernel(user_ids, user_tags_idx, item_ids, content_features, user_table,
           tag_table, item_table, tp_W, tp_b, tp_g, tp_beta, uf_W, uf_b, uf_g,
           uf_beta, cf_W, cf_b, cf_g, cf_beta, if_W, if_b, if_g, if_beta,
           m1_W, m1_b, m2_W, m2_b, m3_W, m3_b, p_W, p_b):
    uid, tsum, iid = _sc_gather(user_ids, user_tags_idx, item_ids,
                                user_table, tag_table, item_table)

    BM = 1024
    grid = (B // BM,)
    row = pl.BlockSpec((BM, D), lambda i: (i, 0))
    full = lambda a: pl.BlockSpec(a.shape, lambda i: tuple(0 for _ in a.shape))
    weights = [tp_W.T, tp_b.reshape(1, -1), tp_g.reshape(1, -1),
               tp_beta.reshape(1, -1), uf_W.T, uf_b.reshape(1, -1),
               uf_g.reshape(1, -1), uf_beta.reshape(1, -1), cf_W.T,
               cf_b.reshape(1, -1), cf_g.reshape(1, -1),
               cf_beta.reshape(1, -1), if_W.T, if_b.reshape(1, -1),
               if_g.reshape(1, -1), if_beta.reshape(1, -1), m1_W.T,
               m1_b.reshape(1, -1), m2_W.T, m2_b.reshape(1, -1), m3_W.T,
               m3_b.reshape(1, -1), p_W, p_b.reshape(1, -1)]
    out = pl.pallas_call(
        _tc_body,
        grid=grid,
        in_specs=[row, row, row, row] + [full(w) for w in weights],
        out_specs=pl.BlockSpec((BM, 1), lambda i: (i, 0)),
        out_shape=jax.ShapeDtypeStruct((B, 1), jnp.float32),
    )(uid, tsum, iid, content_features, *weights)
    return out.reshape(B)
